# baseline (speedup 1.0000x reference)
import functools

import jax
import jax.numpy as jnp
from jax import lax
from jax.experimental import pallas as pl
from jax.experimental.pallas import tpu as pltpu
from jax.experimental.pallas import tpu_sc as plsc

_N = 10000
_E = 320000
_D = 128
_NC = 2
_NS = 16
_NW = _NC * _NS
_EPW = _E // _NW
_K = 80
_NCH = _EPW // _K
_STRIPE = 624
_LAST = _N - (_NS - 1) * _STRIPE
_DEGW = 128

_sc_mesh = plsc.VectorSubcoreMesh(core_axis_name="c", subcore_axis_name="s")


def _stripe_io(s, copy_fn_main, copy_fn_last):
    @pl.when(s < _NS - 1)
    def _():
        copy_fn_main()

    @pl.when(s == _NS - 1)
    def _():
        copy_fn_last()


_NBUF = 3


@functools.partial(
    pl.kernel,
    mesh=_sc_mesh,
    out_type=jax.ShapeDtypeStruct((_NC, _N, _DEGW), jnp.float32),
    scratch_types=[
        pltpu.VMEM((_NCH, _K), jnp.int32),
        pltpu.VMEM((_K, _DEGW), jnp.float32),
        pltpu.VMEM_SHARED((_N, _DEGW), jnp.float32),
        pltpu.SemaphoreType.DMA((_NBUF,)),
    ],
)
def _sc_degree(dst_hbm, zeros_hbm, ones_hbm, out_hbm, didx_v, ones_v, deg_sh,
               ssem):
    c = lax.axis_index("c")
    s = lax.axis_index("s")
    wid = c * _NS + s
    pltpu.sync_copy(ones_hbm, ones_v)
    pltpu.sync_copy(dst_hbm.at[wid], didx_v)
    base = s * _STRIPE
    _stripe_io(
        s,
        lambda: pltpu.sync_copy(zeros_hbm.at[pl.ds(0, _STRIPE)],
                                deg_sh.at[pl.ds(base, _STRIPE)]),
        lambda: pltpu.sync_copy(zeros_hbm, deg_sh.at[pl.ds(base, _LAST)]),
    )
    plsc.subcore_barrier()

    def scat(j, b):
        pltpu.make_async_copy(ones_v, deg_sh.at[didx_v.at[j]],
                              ssem.at[b]).start(add=True)

    def wait_scat(b):
        pltpu.make_async_copy(ones_v, deg_sh.at[didx_v.at[0]],
                              ssem.at[b]).wait()

    n = _NBUF
    for b in range(n):
        scat(b, b)
    q = (_NCH - n) // n

    def body(jj, carry):
        for i in range(n):
            j = n + jj * n + i
            wait_scat(i)
            scat(j, i)
        return carry

    lax.fori_loop(0, q, body, 0)
    for j in range(n + q * n, _NCH):
        wait_scat(j % n)
        scat(j, j % n)
    for j in range(_NCH - n, _NCH):
        wait_scat(j % n)
    plsc.subcore_barrier()
    _stripe_io(
        s,
        lambda: pltpu.sync_copy(deg_sh.at[pl.ds(base, _STRIPE)],
                                out_hbm.at[c, pl.ds(base, _STRIPE)]),
        lambda: pltpu.sync_copy(deg_sh.at[pl.ds(base, _LAST)],
                                out_hbm.at[c, pl.ds(base, _LAST)]),
    )


_MK = 40
_WIN = 5
_NGRP = _EPW // (_WIN * _MK)


@functools.partial(
    pl.kernel,
    mesh=_sc_mesh,
    out_type=jax.ShapeDtypeStruct((_NC, _N, _D), jnp.float32),
    scratch_types=[
        pltpu.VMEM((2, _WIN, _MK), jnp.int32),
        pltpu.VMEM((2, _WIN, _MK), jnp.int32),
        pltpu.VMEM((_WIN, _MK, _D), jnp.float32),
        pltpu.VMEM_SHARED((_N, _D), jnp.float32),
        pltpu.SemaphoreType.DMA((_WIN,)),
        pltpu.SemaphoreType.DMA((_WIN,)),
        pltpu.SemaphoreType.DMA((2,)),
    ],
)
def _sc_scatter(hs_hbm, src_hbm, dst_hbm, zeros_hbm, out_hbm,
                sidw, didw, rows_v, acc_sh, gsem, ssem, isem):
    c = lax.axis_index("c")
    s = lax.axis_index("s")
    wid = c * _NS + s
    base = s * _STRIPE

    def fill(g, p):
        pltpu.make_async_copy(src_hbm.at[wid, g], sidw.at[p],
                              isem.at[p]).start()
        pltpu.make_async_copy(dst_hbm.at[wid, g], didw.at[p],
                              isem.at[p]).start()

    def wait_fill(p):
        pltpu.make_async_copy(src_hbm.at[wid, 0], sidw.at[p],
                              isem.at[p]).wait()
        pltpu.make_async_copy(dst_hbm.at[wid, 0], didw.at[p],
                              isem.at[p]).wait()

    def gat(p, i, b):
        pltpu.make_async_copy(hs_hbm.at[sidw.at[p, i]], rows_v.at[b],
                              gsem.at[b]).start()

    def wait_gat(b):
        pltpu.make_async_copy(hs_hbm.at[sidw.at[0, 0]], rows_v.at[b],
                              gsem.at[b]).wait()

    def scat(p, i, b):
        pltpu.make_async_copy(rows_v.at[b], acc_sh.at[didw.at[p, i]],
                              ssem.at[b]).start(add=True)

    def wait_scat(b):
        pltpu.make_async_copy(rows_v.at[b], acc_sh.at[didw.at[0, 0]],
                              ssem.at[b]).wait()

    def group(g, p, first=False, last=False):
        for i in range(_WIN):
            wait_gat(i)
            scat(p, i, i)
            if not (first and i < _WIN - 1):
                wait_scat((i + 1) % _WIN)
            if i == _WIN - 2 and not first and not last:
                fill(g + 1, 1 - p)
            if i < _WIN - 1:
                gat(p, i + 1, i + 1)
            elif not last:
                wait_fill(1 - p)
                gat(1 - p, 0, 0)

    fill(0, 0)
    fill(1, 1)
    _stripe_io(
        s,
        lambda: pltpu.sync_copy(zeros_hbm.at[pl.ds(0, _STRIPE)],
                                acc_sh.at[pl.ds(base, _STRIPE)]),
        lambda: pltpu.sync_copy(zeros_hbm, acc_sh.at[pl.ds(base, _LAST)]),
    )
    plsc.subcore_barrier()
    wait_fill(0)
    gat(0, 0, 0)
    group(0, 0, first=True)

    def body(jj, carry):
        group(1 + 2 * jj, 1)
        group(2 + 2 * jj, 0)
        return carry

    lax.fori_loop(0, (_NGRP - 2) // 2, body, 0)
    group(_NGRP - 1, (_NGRP - 1) % 2, last=True)
    for b in range(1, _WIN):
        wait_scat(b)
    plsc.subcore_barrier()
    _stripe_io(
        s,
        lambda: pltpu.sync_copy(acc_sh.at[pl.ds(base, _STRIPE)],
                                out_hbm.at[c, pl.ds(base, _STRIPE)]),
        lambda: pltpu.sync_copy(acc_sh.at[pl.ds(base, _LAST)],
                                out_hbm.at[c, pl.ds(base, _LAST)]),
    )


_B = 2000


def _dinv_block(degp_ref):
    deg = 1.0 + degp_ref[0] + degp_ref[1]
    return lax.rsqrt(deg)[:, 0:1]


def _tc_mm_scale_body(degp_ref, x_ref, w_ref, out_ref):
    out_ref[...] = jnp.dot(x_ref[...], w_ref[...],
                           preferred_element_type=jnp.float32) * _dinv_block(degp_ref)


_tc_mm_scale = pl.pallas_call(
    _tc_mm_scale_body,
    grid=(_N // _B,),
    in_specs=[
        pl.BlockSpec((_NC, _B, _DEGW), lambda i: (0, i, 0)),
        pl.BlockSpec((_B, _D), lambda i: (i, 0)),
        pl.BlockSpec((_D, _D), lambda i: (0, 0)),
    ],
    out_specs=pl.BlockSpec((_B, _D), lambda i: (i, 0)),
    out_shape=jax.ShapeDtypeStruct((_N, _D), jnp.float32),
)


def _tc_mid_body(degp_ref, acc_ref, hs_ref, b_ref, w_ref, out_ref):
    dinv = _dinv_block(degp_ref)
    pre = (acc_ref[0] + acc_ref[1] + hs_ref[...]) * dinv + b_ref[...]
    x2 = jnp.maximum(pre, 0.0)
    out_ref[...] = jnp.dot(x2, w_ref[...],
                           preferred_element_type=jnp.float32) * dinv


_tc_mid = pl.pallas_call(
    _tc_mid_body,
    grid=(_N // _B,),
    in_specs=[
        pl.BlockSpec((_NC, _B, _DEGW), lambda i: (0, i, 0)),
        pl.BlockSpec((_NC, _B, _D), lambda i: (0, i, 0)),
        pl.BlockSpec((_B, _D), lambda i: (i, 0)),
        pl.BlockSpec((1, _D), lambda i: (0, 0)),
        pl.BlockSpec((_D, _D), lambda i: (0, 0)),
    ],
    out_specs=pl.BlockSpec((_B, _D), lambda i: (i, 0)),
    out_shape=jax.ShapeDtypeStruct((_N, _D), jnp.float32),
)


def _tc_final_body(degp_ref, acc_ref, hs_ref, b_ref, out_ref):
    dinv = _dinv_block(degp_ref)
    out_ref[...] = (acc_ref[0] + acc_ref[1] + hs_ref[...]) * dinv + b_ref[...]


_tc_final = pl.pallas_call(
    _tc_final_body,
    grid=(_N // _B,),
    in_specs=[
        pl.BlockSpec((_NC, _B, _DEGW), lambda i: (0, i, 0)),
        pl.BlockSpec((_NC, _B, _D), lambda i: (0, i, 0)),
        pl.BlockSpec((_B, _D), lambda i: (i, 0)),
        pl.BlockSpec((1, _D), lambda i: (0, 0)),
    ],
    out_specs=pl.BlockSpec((_B, _D), lambda i: (i, 0)),
    out_shape=jax.ShapeDtypeStruct((_N, _D), jnp.float32),
)


def kernel(x, edge_index, batch, W1, b1, W2, b2):
    dst3 = edge_index[1].reshape(_NW, _NCH, _K)
    src4 = edge_index[0].reshape(_NW, _NGRP, _WIN, _MK)
    dst4 = edge_index[1].reshape(_NW, _NGRP, _WIN, _MK)
    zeros_deg = jnp.zeros((_LAST, _DEGW), jnp.float32)
    ones_deg = jnp.ones((_K, _DEGW), jnp.float32)
    zeros_acc = jnp.zeros((_LAST, _D), jnp.float32)

    degp = _sc_degree(dst3, zeros_deg, ones_deg)
    hs1 = _tc_mm_scale(degp, x, W1)
    acc1 = _sc_scatter(hs1, src4, dst4, zeros_acc)
    hs2 = _tc_mid(degp, acc1, hs1, b1.reshape(1, _D), W2)
    acc2 = _sc_scatter(hs2, src4, dst4, zeros_acc)
    out = _tc_final(degp, acc2, hs2, b2.reshape(1, _D))
    return out


# speedup vs baseline: 1.5145x; 1.5145x over previous
import functools

import jax
import jax.numpy as jnp
from jax import lax
from jax.experimental import pallas as pl
from jax.experimental.pallas import tpu as pltpu
from jax.experimental.pallas import tpu_sc as plsc

_N = 10000
_E = 320000
_D = 128
_NC = 2
_NS = 16
_NW = _NC * _NS
_EPW = _E // _NW
_K = 125
_NCH = _EPW // _K
_STRIPE = 624
_LAST = _N - (_NS - 1) * _STRIPE
_DEGW = 128

_sc_mesh = plsc.VectorSubcoreMesh(core_axis_name="c", subcore_axis_name="s")


def _stripe_io(s, copy_fn_main, copy_fn_last):
    @pl.when(s < _NS - 1)
    def _():
        copy_fn_main()

    @pl.when(s == _NS - 1)
    def _():
        copy_fn_last()


_NBUF = 4


@functools.partial(
    pl.kernel,
    mesh=_sc_mesh,
    out_type=jax.ShapeDtypeStruct((_NC, _N, _DEGW), jnp.float32),
    scratch_types=[
        pltpu.VMEM((_NCH, _K), jnp.int32),
        pltpu.VMEM((_K, _DEGW), jnp.float32),
        pltpu.VMEM_SHARED((_N, _DEGW), jnp.float32),
        pltpu.SemaphoreType.DMA((_NBUF,)),
    ],
)
def _sc_degree(dst_hbm, zeros_hbm, ones_hbm, out_hbm, didx_v, ones_v, deg_sh,
               ssem):
    c = lax.axis_index("c")
    s = lax.axis_index("s")
    wid = c * _NS + s
    pltpu.sync_copy(ones_hbm, ones_v)
    pltpu.sync_copy(dst_hbm.at[wid], didx_v)
    base = s * _STRIPE
    _stripe_io(
        s,
        lambda: pltpu.sync_copy(zeros_hbm.at[pl.ds(0, _STRIPE)],
                                deg_sh.at[pl.ds(base, _STRIPE)]),
        lambda: pltpu.sync_copy(zeros_hbm, deg_sh.at[pl.ds(base, _LAST)]),
    )
    plsc.subcore_barrier()

    def scat(j, b):
        pltpu.make_async_copy(ones_v, deg_sh.at[didx_v.at[j]],
                              ssem.at[b]).start(add=True)

    def wait_scat(b):
        pltpu.make_async_copy(ones_v, deg_sh.at[didx_v.at[0]],
                              ssem.at[b]).wait()

    n = _NBUF
    for b in range(n):
        scat(b, b)
    q = (_NCH - n) // n

    def body(jj, carry):
        for i in range(n):
            j = n + jj * n + i
            wait_scat(i)
            scat(j, i)
        return carry

    lax.fori_loop(0, q, body, 0)
    for j in range(n + q * n, _NCH):
        wait_scat(j % n)
        scat(j, j % n)
    for j in range(_NCH - n, _NCH):
        wait_scat(j % n)
    plsc.subcore_barrier()
    _stripe_io(
        s,
        lambda: pltpu.sync_copy(deg_sh.at[pl.ds(base, _STRIPE)],
                                out_hbm.at[c, pl.ds(base, _STRIPE)]),
        lambda: pltpu.sync_copy(deg_sh.at[pl.ds(base, _LAST)],
                                out_hbm.at[c, pl.ds(base, _LAST)]),
    )


_WIN = 4
_NGRP = _NCH // _WIN
_NB = 2


@functools.partial(
    pl.kernel,
    mesh=_sc_mesh,
    out_type=jax.ShapeDtypeStruct((_NC, _N, _D), jnp.float32),
    scratch_types=[
        pltpu.VMEM((2, _WIN, _K), jnp.int32),
        pltpu.VMEM((2, _WIN, _K), jnp.int32),
        pltpu.VMEM((_NB, _K, _D), jnp.float32),
        pltpu.VMEM_SHARED((_N, _D), jnp.float32),
        pltpu.SemaphoreType.DMA((_NB,)),
        pltpu.SemaphoreType.DMA((_NB,)),
        pltpu.SemaphoreType.DMA((2,)),
    ],
)
def _sc_scatter(hs_hbm, src_hbm, dst_hbm, zeros_hbm, out_hbm,
                sidw, didw, rows_v, acc_sh, gsem, ssem, isem):
    c = lax.axis_index("c")
    s = lax.axis_index("s")
    wid = c * _NS + s
    base = s * _STRIPE

    def fill(g, p):
        pltpu.make_async_copy(src_hbm.at[wid, g], sidw.at[p],
                              isem.at[p]).start()
        pltpu.make_async_copy(dst_hbm.at[wid, g], didw.at[p],
                              isem.at[p]).start()

    def wait_fill(p):
        pltpu.make_async_copy(src_hbm.at[wid, 0], sidw.at[p],
                              isem.at[p]).wait()
        pltpu.make_async_copy(dst_hbm.at[wid, 0], didw.at[p],
                              isem.at[p]).wait()

    def gat(p, i, b):
        pltpu.make_async_copy(hs_hbm.at[sidw.at[p, i]], rows_v.at[b],
                              gsem.at[b]).start()

    def wait_gat(b):
        pltpu.make_async_copy(hs_hbm.at[sidw.at[0, 0]], rows_v.at[b],
                              gsem.at[b]).wait()

    def scat(p, i, b):
        pltpu.make_async_copy(rows_v.at[b], acc_sh.at[didw.at[p, i]],
                              ssem.at[b]).start(add=True)

    def wait_scat(b):
        pltpu.make_async_copy(rows_v.at[b], acc_sh.at[didw.at[0, 0]],
                              ssem.at[b]).wait()

    def group(g, p, first=False, last=False):
        for i in range(_WIN):
            b = i % _NB
            wait_gat(b)
            scat(p, i, b)
            if not (first and i == 0):
                wait_scat(1 - b)
            if i == 0 and not first and not last:
                fill(g + 1, 1 - p)
            if i < _WIN - 1:
                gat(p, i + 1, 1 - b)
            elif not last:
                wait_fill(1 - p)
                gat(1 - p, 0, 0)

    fill(0, 0)
    fill(1, 1)
    _stripe_io(
        s,
        lambda: pltpu.sync_copy(zeros_hbm.at[pl.ds(0, _STRIPE)],
                                acc_sh.at[pl.ds(base, _STRIPE)]),
        lambda: pltpu.sync_copy(zeros_hbm, acc_sh.at[pl.ds(base, _LAST)]),
    )
    plsc.subcore_barrier()
    wait_fill(0)
    gat(0, 0, 0)
    group(0, 0, first=True)

    def body(jj, carry):
        group(1 + 2 * jj, 1)
        group(2 + 2 * jj, 0)
        return carry

    lax.fori_loop(0, (_NGRP - 2) // 2, body, 0)
    group(_NGRP - 1, (_NGRP - 1) % 2, last=True)
    wait_scat((_NCH - 1) % _NB)
    plsc.subcore_barrier()
    _stripe_io(
        s,
        lambda: pltpu.sync_copy(acc_sh.at[pl.ds(base, _STRIPE)],
                                out_hbm.at[c, pl.ds(base, _STRIPE)]),
        lambda: pltpu.sync_copy(acc_sh.at[pl.ds(base, _LAST)],
                                out_hbm.at[c, pl.ds(base, _LAST)]),
    )


_B = 2000


def _dinv_block(degp_ref):
    deg = 1.0 + degp_ref[0] + degp_ref[1]
    return lax.rsqrt(deg)[:, 0:1]


def _tc_mm_scale_body(degp_ref, x_ref, w_ref, out_ref):
    out_ref[...] = jnp.dot(x_ref[...], w_ref[...],
                           preferred_element_type=jnp.float32) * _dinv_block(degp_ref)


_tc_mm_scale = pl.pallas_call(
    _tc_mm_scale_body,
    grid=(_N // _B,),
    in_specs=[
        pl.BlockSpec((_NC, _B, _DEGW), lambda i: (0, i, 0)),
        pl.BlockSpec((_B, _D), lambda i: (i, 0)),
        pl.BlockSpec((_D, _D), lambda i: (0, 0)),
    ],
    out_specs=pl.BlockSpec((_B, _D), lambda i: (i, 0)),
    out_shape=jax.ShapeDtypeStruct((_N, _D), jnp.float32),
)


def _tc_mid_body(degp_ref, acc_ref, hs_ref, b_ref, w_ref, out_ref):
    dinv = _dinv_block(degp_ref)
    pre = (acc_ref[0] + acc_ref[1] + hs_ref[...]) * dinv + b_ref[...]
    x2 = jnp.maximum(pre, 0.0)
    out_ref[...] = jnp.dot(x2, w_ref[...],
                           preferred_element_type=jnp.float32) * dinv


_tc_mid = pl.pallas_call(
    _tc_mid_body,
    grid=(_N // _B,),
    in_specs=[
        pl.BlockSpec((_NC, _B, _DEGW), lambda i: (0, i, 0)),
        pl.BlockSpec((_NC, _B, _D), lambda i: (0, i, 0)),
        pl.BlockSpec((_B, _D), lambda i: (i, 0)),
        pl.BlockSpec((1, _D), lambda i: (0, 0)),
        pl.BlockSpec((_D, _D), lambda i: (0, 0)),
    ],
    out_specs=pl.BlockSpec((_B, _D), lambda i: (i, 0)),
    out_shape=jax.ShapeDtypeStruct((_N, _D), jnp.float32),
)


def _tc_final_body(degp_ref, acc_ref, hs_ref, b_ref, out_ref):
    dinv = _dinv_block(degp_ref)
    out_ref[...] = (acc_ref[0] + acc_ref[1] + hs_ref[...]) * dinv + b_ref[...]


_tc_final = pl.pallas_call(
    _tc_final_body,
    grid=(_N // _B,),
    in_specs=[
        pl.BlockSpec((_NC, _B, _DEGW), lambda i: (0, i, 0)),
        pl.BlockSpec((_NC, _B, _D), lambda i: (0, i, 0)),
        pl.BlockSpec((_B, _D), lambda i: (i, 0)),
        pl.BlockSpec((1, _D), lambda i: (0, 0)),
    ],
    out_specs=pl.BlockSpec((_B, _D), lambda i: (i, 0)),
    out_shape=jax.ShapeDtypeStruct((_N, _D), jnp.float32),
)


def kernel(x, edge_index, batch, W1, b1, W2, b2):
    dst3 = edge_index[1].reshape(_NW, _NCH, _K)
    src4 = edge_index[0].reshape(_NW, _NGRP, _WIN, _K)
    dst4 = edge_index[1].reshape(_NW, _NGRP, _WIN, _K)
    zeros_deg = jnp.zeros((_LAST, _DEGW), jnp.float32)
    ones_deg = jnp.ones((_K, _DEGW), jnp.float32)
    zeros_acc = jnp.zeros((_LAST, _D), jnp.float32)

    degp = _sc_degree(dst3, zeros_deg, ones_deg)
    hs1 = _tc_mm_scale(degp, x, W1)
    acc1 = _sc_scatter(hs1, src4, dst4, zeros_acc)
    hs2 = _tc_mid(degp, acc1, hs1, b1.reshape(1, _D), W2)
    acc2 = _sc_scatter(hs2, src4, dst4, zeros_acc)
    out = _tc_final(degp, acc2, hs2, b2.reshape(1, _D))
    return out


# speedup vs baseline: 1.7777x; 1.1738x over previous
import functools

import jax
import jax.numpy as jnp
from jax import lax
from jax.experimental import pallas as pl
from jax.experimental.pallas import tpu as pltpu
from jax.experimental.pallas import tpu_sc as plsc

_N = 10000
_E = 320000
_D = 128
_NC = 2
_NS = 16
_NW = _NC * _NS
_EPW = _E // _NW
_K = 125
_NCH = _EPW // _K
_STRIPE = 624
_LAST = _N - (_NS - 1) * _STRIPE
_DEGW = 128

_sc_mesh = plsc.VectorSubcoreMesh(core_axis_name="c", subcore_axis_name="s")


def _stripe_io(s, copy_fn_main, copy_fn_last):
    @pl.when(s < _NS - 1)
    def _():
        copy_fn_main()

    @pl.when(s == _NS - 1)
    def _():
        copy_fn_last()


_NBUF = 4


@functools.partial(
    pl.kernel,
    mesh=_sc_mesh,
    out_type=jax.ShapeDtypeStruct((_NC, _N, _DEGW), jnp.float32),
    scratch_types=[
        pltpu.VMEM((_NCH, _K), jnp.int32),
        pltpu.VMEM((_K, _DEGW), jnp.float32),
        pltpu.VMEM_SHARED((_N, _DEGW), jnp.float32),
        pltpu.SemaphoreType.DMA((_NBUF,)),
    ],
)
def _sc_degree(dst_hbm, zeros_hbm, ones_hbm, out_hbm, didx_v, ones_v, deg_sh,
               ssem):
    c = lax.axis_index("c")
    s = lax.axis_index("s")
    wid = c * _NS + s
    pltpu.sync_copy(ones_hbm, ones_v)
    pltpu.sync_copy(dst_hbm.at[wid], didx_v)
    base = s * _STRIPE
    _stripe_io(
        s,
        lambda: pltpu.sync_copy(zeros_hbm.at[pl.ds(0, _STRIPE)],
                                deg_sh.at[pl.ds(base, _STRIPE)]),
        lambda: pltpu.sync_copy(zeros_hbm, deg_sh.at[pl.ds(base, _LAST)]),
    )
    plsc.subcore_barrier()

    def scat(j, b):
        pltpu.make_async_copy(ones_v, deg_sh.at[didx_v.at[j]],
                              ssem.at[b]).start(add=True)

    def wait_scat(b):
        pltpu.make_async_copy(ones_v, deg_sh.at[didx_v.at[0]],
                              ssem.at[b]).wait()

    n = _NBUF
    for b in range(n):
        scat(b, b)
    q = (_NCH - n) // n

    def body(jj, carry):
        for i in range(n):
            j = n + jj * n + i
            wait_scat(i)
            scat(j, i)
        return carry

    lax.fori_loop(0, q, body, 0)
    for j in range(n + q * n, _NCH):
        wait_scat(j % n)
        scat(j, j % n)
    for j in range(_NCH - n, _NCH):
        wait_scat(j % n)
    plsc.subcore_barrier()
    _stripe_io(
        s,
        lambda: pltpu.sync_copy(deg_sh.at[pl.ds(base, _STRIPE)],
                                out_hbm.at[c, pl.ds(base, _STRIPE)]),
        lambda: pltpu.sync_copy(deg_sh.at[pl.ds(base, _LAST)],
                                out_hbm.at[c, pl.ds(base, _LAST)]),
    )


_MK = 100
_MNCH = _EPW // _MK
_WIN = 5
_NGRP = _MNCH // _WIN
_NB = 3


@functools.partial(
    pl.kernel,
    mesh=_sc_mesh,
    out_type=jax.ShapeDtypeStruct((_NC, _N, _D), jnp.float32),
    scratch_types=[
        pltpu.VMEM((2, 2 * _WIN, _MK), jnp.int32),
        pltpu.VMEM((_NB, _MK, _D), jnp.float32),
        pltpu.VMEM_SHARED((_N, _D), jnp.float32),
        pltpu.SemaphoreType.DMA((_NB,)),
        pltpu.SemaphoreType.DMA((_NB,)),
        pltpu.SemaphoreType.DMA((2,)),
    ],
)
def _sc_scatter(hs_hbm, sd_hbm, zeros_hbm, out_hbm,
                sdw, rows_v, acc_sh, gsem, ssem, isem):
    c = lax.axis_index("c")
    s = lax.axis_index("s")
    wid = c * _NS + s
    base = s * _STRIPE

    def fill(g, p):
        pltpu.make_async_copy(sd_hbm.at[wid, g], sdw.at[p],
                              isem.at[p]).start()

    def wait_fill(p):
        pltpu.make_async_copy(sd_hbm.at[wid, 0], sdw.at[p],
                              isem.at[p]).wait()

    def gat(p, i, b):
        pltpu.make_async_copy(hs_hbm.at[sdw.at[p, i]], rows_v.at[b],
                              gsem.at[b]).start()

    def wait_gat(b):
        pltpu.make_async_copy(hs_hbm.at[sdw.at[0, 0]], rows_v.at[b],
                              gsem.at[b]).wait()

    def scat(p, i, b):
        pltpu.make_async_copy(rows_v.at[b], acc_sh.at[sdw.at[p, _WIN + i]],
                              ssem.at[b]).start(add=True)

    def wait_scat(b):
        pltpu.make_async_copy(rows_v.at[b], acc_sh.at[sdw.at[0, _WIN]],
                              ssem.at[b]).wait()

    def group(g, p, boff, first=False, last=False):
        for i in range(_WIN):
            b = (boff + i) % _NB
            wait_gat(b)
            scat(p, i, b)
            if not (first and i == 0):
                wait_scat((b + 2) % _NB)
            if i == 0 and not first and not last:
                fill(g + 1, 1 - p)
            if i < _WIN - 2:
                gat(p, i + 2, (b + 2) % _NB)
            elif not last:
                if i == _WIN - 2:
                    wait_fill(1 - p)
                gat(1 - p, i + 2 - _WIN, (b + 2) % _NB)

    fill(0, 0)
    fill(1, 1)
    _stripe_io(
        s,
        lambda: pltpu.sync_copy(zeros_hbm.at[pl.ds(0, _STRIPE)],
                                acc_sh.at[pl.ds(base, _STRIPE)]),
        lambda: pltpu.sync_copy(zeros_hbm, acc_sh.at[pl.ds(base, _LAST)]),
    )
    plsc.subcore_barrier()
    wait_fill(0)
    gat(0, 0, 0)
    gat(0, 1, 1)
    group(0, 0, 0, first=True)

    def body(jj, carry):
        for k in range(6):
            g = 1 + 6 * jj + k
            group(g, (1 + k) % 2, (_WIN * (1 + k)) % _NB)
        return carry

    lax.fori_loop(0, (_NGRP - 2) // 6, body, 0)
    group(_NGRP - 1, (_NGRP - 1) % 2, (_WIN * (_NGRP - 1)) % _NB, last=True)
    wait_scat((_MNCH - 1) % _NB)
    plsc.subcore_barrier()
    _stripe_io(
        s,
        lambda: pltpu.sync_copy(acc_sh.at[pl.ds(base, _STRIPE)],
                                out_hbm.at[c, pl.ds(base, _STRIPE)]),
        lambda: pltpu.sync_copy(acc_sh.at[pl.ds(base, _LAST)],
                                out_hbm.at[c, pl.ds(base, _LAST)]),
    )


_B = 2000


def _dinv_block(degp_ref):
    deg = 1.0 + degp_ref[0] + degp_ref[1]
    return lax.rsqrt(deg)[:, 0:1]


def _tc_mm_scale_body(degp_ref, x_ref, w_ref, out_ref):
    out_ref[...] = jnp.dot(x_ref[...], w_ref[...],
                           preferred_element_type=jnp.float32) * _dinv_block(degp_ref)


_tc_mm_scale = pl.pallas_call(
    _tc_mm_scale_body,
    grid=(_N // _B,),
    in_specs=[
        pl.BlockSpec((_NC, _B, _DEGW), lambda i: (0, i, 0)),
        pl.BlockSpec((_B, _D), lambda i: (i, 0)),
        pl.BlockSpec((_D, _D), lambda i: (0, 0)),
    ],
    out_specs=pl.BlockSpec((_B, _D), lambda i: (i, 0)),
    out_shape=jax.ShapeDtypeStruct((_N, _D), jnp.float32),
)


def _tc_mid_body(degp_ref, acc_ref, hs_ref, b_ref, w_ref, out_ref):
    dinv = _dinv_block(degp_ref)
    pre = (acc_ref[0] + acc_ref[1] + hs_ref[...]) * dinv + b_ref[...]
    x2 = jnp.maximum(pre, 0.0)
    out_ref[...] = jnp.dot(x2, w_ref[...],
                           preferred_element_type=jnp.float32) * dinv


_tc_mid = pl.pallas_call(
    _tc_mid_body,
    grid=(_N // _B,),
    in_specs=[
        pl.BlockSpec((_NC, _B, _DEGW), lambda i: (0, i, 0)),
        pl.BlockSpec((_NC, _B, _D), lambda i: (0, i, 0)),
        pl.BlockSpec((_B, _D), lambda i: (i, 0)),
        pl.BlockSpec((1, _D), lambda i: (0, 0)),
        pl.BlockSpec((_D, _D), lambda i: (0, 0)),
    ],
    out_specs=pl.BlockSpec((_B, _D), lambda i: (i, 0)),
    out_shape=jax.ShapeDtypeStruct((_N, _D), jnp.float32),
)


def _tc_final_body(degp_ref, acc_ref, hs_ref, b_ref, out_ref):
    dinv = _dinv_block(degp_ref)
    out_ref[...] = (acc_ref[0] + acc_ref[1] + hs_ref[...]) * dinv + b_ref[...]


_tc_final = pl.pallas_call(
    _tc_final_body,
    grid=(_N // _B,),
    in_specs=[
        pl.BlockSpec((_NC, _B, _DEGW), lambda i: (0, i, 0)),
        pl.BlockSpec((_NC, _B, _D), lambda i: (0, i, 0)),
        pl.BlockSpec((_B, _D), lambda i: (i, 0)),
        pl.BlockSpec((1, _D), lambda i: (0, 0)),
    ],
    out_specs=pl.BlockSpec((_B, _D), lambda i: (i, 0)),
    out_shape=jax.ShapeDtypeStruct((_N, _D), jnp.float32),
)


def kernel(x, edge_index, batch, W1, b1, W2, b2):
    dst3 = edge_index[1].reshape(_NW, _NCH, _K)
    sd4 = jnp.concatenate(
        [edge_index[0].reshape(_NW, _NGRP, _WIN, _MK),
         edge_index[1].reshape(_NW, _NGRP, _WIN, _MK)], axis=2)
    zeros_deg = jnp.zeros((_LAST, _DEGW), jnp.float32)
    ones_deg = jnp.ones((_K, _DEGW), jnp.float32)
    zeros_acc = jnp.zeros((_LAST, _D), jnp.float32)

    degp = _sc_degree(dst3, zeros_deg, ones_deg)
    hs1 = _tc_mm_scale(degp, x, W1)
    acc1 = _sc_scatter(hs1, sd4, zeros_acc)
    hs2 = _tc_mid(degp, acc1, hs1, b1.reshape(1, _D), W2)
    acc2 = _sc_scatter(hs2, sd4, zeros_acc)
    out = _tc_final(degp, acc2, hs2, b2.reshape(1, _D))
    return out
